# Initial kernel scaffold; baseline (speedup 1.0000x reference)
import jax
import jax.numpy as jnp
from jax.experimental import pallas as pl


def kernel(x, edge_index, batch, W0, a_src0, a_dst0, b0, lw0, lb0, W1, a_src1, a_dst1, b1, lw1, lb1):
    raise NotImplementedError("write your pallas kernel here")


# speedup vs baseline: 41.2920x; 41.2920x over previous
import dataclasses
import functools

import jax
import jax.numpy as jnp
from jax import lax
from jax.experimental import pallas as pl
from jax.experimental.pallas import tpu as pltpu
from jax.experimental.pallas import tpu_sc as plsc

F = 128
H = 4
C = 32
NP = 10240
TW = 80
T1W = 16
CHUNK = 128
NW = 32


def _f16(v):
    return jnp.full((16,), v, jnp.int32)


def _sc_params():
    cp = pltpu.CompilerParams()
    for field, val in (("needs_layout_passes", False),
                       ("use_tc_tiling_on_sc", False)):
        if field in pltpu.CompilerParams.__dataclass_fields__:
            cp = dataclasses.replace(cp, **{field: val})
    return cp


def _prep0_body(x_ref, w_ref, lw_ref, am_ref, bsk_ref, ta_ref, tb_ref, a_ref, s_ref):
    xb = x_ref[...]
    h = jnp.dot(xb, w_ref[...], preferred_element_type=jnp.float32)
    aa = jnp.dot(h, am_ref[...], preferred_element_type=jnp.float32)
    zp = jnp.zeros((xb.shape[0], 14), jnp.float32)
    ta_ref[...] = jnp.concatenate([h[:, 0:64], aa[:, 0:2], zp], axis=1)
    tb_ref[...] = jnp.concatenate([h[:, 64:128], aa[:, 2:4], zp], axis=1)
    a_ref[...] = jnp.concatenate([aa[:, 4:8], zp[:, 0:12]], axis=1)
    s_ref[...] = jnp.dot(xb, lw_ref[...], preferred_element_type=jnp.float32) + bsk_ref[...]


def _prep0(x_p, w0, lw0, am, bsk):
    bn = 1024
    return pl.pallas_call(
        _prep0_body,
        grid=(NP // bn,),
        in_specs=[
            pl.BlockSpec((bn, F), lambda i: (i, 0)),
            pl.BlockSpec((F, F), lambda i: (0, 0)),
            pl.BlockSpec((F, F), lambda i: (0, 0)),
            pl.BlockSpec((F, 8), lambda i: (0, 0)),
            pl.BlockSpec((1, F), lambda i: (0, 0)),
        ],
        out_specs=[
            pl.BlockSpec((bn, TW), lambda i: (i, 0)),
            pl.BlockSpec((bn, TW), lambda i: (i, 0)),
            pl.BlockSpec((bn, 16), lambda i: (i, 0)),
            pl.BlockSpec((bn, F), lambda i: (i, 0)),
        ],
        out_shape=[
            jax.ShapeDtypeStruct((NP, TW), jnp.float32),
            jax.ShapeDtypeStruct((NP, TW), jnp.float32),
            jax.ShapeDtypeStruct((NP, 16), jnp.float32),
            jax.ShapeDtypeStruct((NP, F), jnp.float32),
        ],
    )(x_p, w0, lw0, am, bsk)


def _edge0_body(ta_hbm, tb_hbm, a0_hbm, src_hbm, dst_hbm, out_hbm,
                srcv, dstv, rowv, av, acc):
    c = lax.axis_index("c")
    s = lax.axis_index("s")
    n_chunks = src_hbm.shape[0]
    per_s = n_chunks // 16
    rows_s = NP // 16

    @pl.loop(0, CHUNK)
    def _(i):
        @pl.loop(0, TW, step=16)
        def _(j):
            rowv[0, i, pl.ds(j, 16)] = jnp.zeros((16,), jnp.float32)

    for k in range(rows_s // CHUNK):
        pltpu.sync_copy(rowv.at[0], acc.at[pl.ds(s * rows_s + k * CHUNK, CHUNK)])
    plsc.subcore_barrier()

    @pl.loop(0, per_s)
    def _(j):
        ch = s * per_s + j
        pltpu.sync_copy(src_hbm.at[ch], srcv.at[0])
        pltpu.sync_copy(dst_hbm.at[ch], dstv.at[0])

        @pl.when(c == 0)
        def _():
            pltpu.sync_copy(ta_hbm.at[srcv.at[0]], rowv.at[0])

        @pl.when(c == 1)
        def _():
            pltpu.sync_copy(tb_hbm.at[srcv.at[0]], rowv.at[0])

        pltpu.sync_copy(a0_hbm.at[dstv.at[0]], av.at[0])

        @pl.loop(0, CHUNK, step=16)
        def _(e0):
            iot = lax.iota(jnp.int32, 16) + e0
            for hl in range(2):
                a_s = plsc.load_gather(rowv.at[0], [iot, _f16(64 + hl)])
                a_d = plsc.load_gather(av.at[0], [iot, _f16(hl) + 2 * c])
                al = a_s + a_d
                al = jnp.where(al > 0, al, al * 0.2)
                ex = jnp.exp(al)
                plsc.store_scatter(rowv.at[0], [iot, _f16(64 + hl)], ex)

        @pl.loop(0, CHUNK)
        def _(e):
            exv = rowv[0, e, pl.ds(64, 16)]
            for hl in range(2):
                sc = exv[hl]
                for v in range(2):
                    off = hl * C + v * 16
                    rowv[0, e, pl.ds(off, 16)] = rowv[0, e, pl.ds(off, 16)] * sc

        pltpu.sync_copy(rowv.at[0], acc.at[dstv.at[0]], add=True)

    plsc.subcore_barrier()
    for k in range(rows_s // CHUNK):
        base = s * rows_s + k * CHUNK
        pltpu.sync_copy(acc.at[pl.ds(base, CHUNK)], out_hbm.at[c, pl.ds(base, CHUNK)])


def _edge0(ta, tb, a0, src2d, dst2d):
    mesh = plsc.VectorSubcoreMesh(core_axis_name="c", subcore_axis_name="s")
    kern = functools.partial(
        pl.kernel,
        mesh=mesh,
        out_type=jax.ShapeDtypeStruct((2, NP, TW), jnp.float32),
        scratch_types=[
            pltpu.VMEM((2, CHUNK), jnp.int32),
            pltpu.VMEM((2, CHUNK), jnp.int32),
            pltpu.VMEM((2, CHUNK, TW), jnp.float32),
            pltpu.VMEM((2, CHUNK, 16), jnp.float32),
            pltpu.VMEM_SHARED((NP, TW), jnp.float32),
        ],
        compiler_params=_sc_params(),
    )(_edge0_body)
    return kern(ta, tb, a0, src2d, dst2d)


def _comb0_body(aA_ref, aB_ref, ta_ref, tb_ref, a0_ref, sk_ref, sel_ref, ar_ref,
                wmix_ref, brow_ref, t1_ref):
    aA = aA_ref[...]
    aB = aB_ref[...]
    h = jnp.concatenate([ta_ref[...][:, 0:64], tb_ref[...][:, 0:64]], axis=1)
    asrc0 = jnp.concatenate([ta_ref[...][:, 64:66], tb_ref[...][:, 64:66]], axis=1)
    als = asrc0 + a0_ref[...][:, 0:4]
    als = jnp.where(als > 0, als, 0.2 * als)
    exs = jnp.exp(als)
    selw = sel_ref[...]
    num = jnp.concatenate([aA[:, 0:64], aB[:, 0:64]], axis=1) \
        + jnp.dot(exs, selw, preferred_element_type=jnp.float32) * h
    den4 = jnp.concatenate([aA[:, 64:66], aB[:, 64:66]], axis=1) + exs
    den = jnp.dot(den4, selw, preferred_element_type=jnp.float32) + 1e-16
    z = num / den + sk_ref[...]
    h1 = jnp.where(z > 0, z, jnp.exp(jnp.minimum(z, 0.0)) - 1.0)
    hwsk = jnp.dot(h1, wmix_ref[...], preferred_element_type=jnp.float32) + brow_ref[...]
    hw1 = hwsk[:, 0:4]
    ar = ar_ref[...]
    zp = jnp.zeros((h.shape[0], 3), jnp.float32)
    t1_ref[...] = jnp.concatenate(
        [hw1, hw1 * ar[:, 0:4], hw1 * ar[:, 4:8], hwsk[:, 4:5], zp], axis=1)


def _comb0(accA, accB, ta, tb, a0, sk0, selw, arow, wmix, brow):
    bn = 1024
    return pl.pallas_call(
        _comb0_body,
        grid=(NP // bn,),
        in_specs=[
            pl.BlockSpec((bn, TW), lambda i: (i, 0)),
            pl.BlockSpec((bn, TW), lambda i: (i, 0)),
            pl.BlockSpec((bn, TW), lambda i: (i, 0)),
            pl.BlockSpec((bn, TW), lambda i: (i, 0)),
            pl.BlockSpec((bn, 16), lambda i: (i, 0)),
            pl.BlockSpec((bn, F), lambda i: (i, 0)),
            pl.BlockSpec((4, F), lambda i: (0, 0)),
            pl.BlockSpec((1, 8), lambda i: (0, 0)),
            pl.BlockSpec((F, 8), lambda i: (0, 0)),
            pl.BlockSpec((1, 8), lambda i: (0, 0)),
        ],
        out_specs=[pl.BlockSpec((bn, T1W), lambda i: (i, 0))],
        out_shape=[jax.ShapeDtypeStruct((NP, T1W), jnp.float32)],
    )(accA, accB, ta, tb, a0, sk0, selw, arow, wmix, brow)[0]


def _edge1_body(t1_hbm, src_hbm, dst_hbm, out_hbm, srcv, dstv, rowv, drow, acc):
    c = lax.axis_index("c")
    s = lax.axis_index("s")
    wid = s * 2 + c
    per_w = src_hbm.shape[0] // NW
    rows_s = NP // 16

    @pl.loop(0, CHUNK)
    def _(i):
        rowv[0, i, pl.ds(0, 16)] = jnp.zeros((16,), jnp.float32)

    for k in range(rows_s // CHUNK):
        pltpu.sync_copy(rowv.at[0], acc.at[pl.ds(s * rows_s + k * CHUNK, CHUNK)])
    plsc.subcore_barrier()

    @pl.loop(0, per_w)
    def _(j):
        ch = wid * per_w + j
        pltpu.sync_copy(src_hbm.at[ch], srcv.at[0])
        pltpu.sync_copy(dst_hbm.at[ch], dstv.at[0])
        pltpu.sync_copy(t1_hbm.at[srcv.at[0]], rowv.at[0])
        pltpu.sync_copy(t1_hbm.at[dstv.at[0]], drow.at[0])

        @pl.loop(0, CHUNK, step=16)
        def _(e0):
            iot = lax.iota(jnp.int32, 16) + e0
            for hd in range(H):
                a_s = plsc.load_gather(rowv.at[0], [iot, _f16(4 + hd)])
                a_d = plsc.load_gather(drow.at[0], [iot, _f16(8 + hd)])
                al = a_s + a_d
                al = jnp.where(al > 0, al, al * 0.2)
                ex = jnp.exp(al)
                hsrc = plsc.load_gather(rowv.at[0], [iot, _f16(hd)])
                plsc.store_scatter(rowv.at[0], [iot, _f16(hd)], hsrc * ex)
                plsc.store_scatter(rowv.at[0], [iot, _f16(4 + hd)], ex)

        pltpu.sync_copy(rowv.at[0], acc.at[dstv.at[0]], add=True)

    plsc.subcore_barrier()
    for k in range(rows_s // CHUNK):
        base = s * rows_s + k * CHUNK
        pltpu.sync_copy(acc.at[pl.ds(base, CHUNK)], out_hbm.at[c, pl.ds(base, CHUNK)])


def _edge1(t1, src2d, dst2d):
    mesh = plsc.VectorSubcoreMesh(core_axis_name="c", subcore_axis_name="s")
    kern = functools.partial(
        pl.kernel,
        mesh=mesh,
        out_type=jax.ShapeDtypeStruct((2, NP, T1W), jnp.float32),
        scratch_types=[
            pltpu.VMEM((2, CHUNK), jnp.int32),
            pltpu.VMEM((2, CHUNK), jnp.int32),
            pltpu.VMEM((2, CHUNK, T1W), jnp.float32),
            pltpu.VMEM((2, CHUNK, T1W), jnp.float32),
            pltpu.VMEM_SHARED((NP, T1W), jnp.float32),
        ],
        compiler_params=_sc_params(),
    )(_edge1_body)
    return kern(t1, src2d, dst2d)


def _final_body(aA_ref, aB_ref, t1_ref, o_ref):
    t1 = t1_ref[...]
    hw1 = t1[:, 0:4]
    als = t1[:, 4:8] + t1[:, 8:12]
    als = jnp.where(als > 0, als, 0.2 * als)
    exs = jnp.exp(als)
    aA = aA_ref[...]
    aB = aB_ref[...]
    num = aA[:, 0:4] + aB[:, 0:4] + exs * hw1
    den = aA[:, 4:8] + aB[:, 4:8] + exs + 1e-16
    o_ref[...] = jnp.mean(num / den, axis=1, keepdims=True) + t1[:, 12:13]


def _final(acc1A, acc1B, t1):
    bn = 512
    return pl.pallas_call(
        _final_body,
        grid=(NP // bn,),
        in_specs=[
            pl.BlockSpec((bn, T1W), lambda i: (i, 0)),
            pl.BlockSpec((bn, T1W), lambda i: (i, 0)),
            pl.BlockSpec((bn, T1W), lambda i: (i, 0)),
        ],
        out_specs=[pl.BlockSpec((bn, 1), lambda i: (i, 0))],
        out_shape=[jax.ShapeDtypeStruct((NP, 1), jnp.float32)],
    )(acc1A, acc1B, t1)[0]


def kernel(x, edge_index, batch, W0, a_src0, a_dst0, b0, lw0, lb0,
           W1, a_src1, a_dst1, b1, lw1, lb1):
    N = x.shape[0]
    E = edge_index.shape[1]
    ep = ((E + NW * CHUNK - 1) // (NW * CHUNK)) * (NW * CHUNK)

    x_p = jnp.pad(x, ((0, NP - N), (0, 0)))
    src = jnp.pad(edge_index[0], (0, ep - E)).reshape(ep // CHUNK, CHUNK)
    dst = jnp.pad(edge_index[1], (0, ep - E), constant_values=N).reshape(ep // CHUNK, CHUNK)

    eyeH = jnp.eye(H, dtype=jnp.float32)
    am = jnp.concatenate([
        jnp.repeat(eyeH, C, axis=0) * a_src0.reshape(-1, 1),
        jnp.repeat(eyeH, C, axis=0) * a_dst0.reshape(-1, 1),
    ], axis=1)
    selw = jnp.repeat(eyeH, C, axis=1)
    bsk = (b0 + lb0).reshape(1, F)
    arow = jnp.concatenate([a_src1[:, 0], a_dst1[:, 0]]).reshape(1, 8)
    wmix = jnp.concatenate([W1, lw1, jnp.zeros((F, 3), jnp.float32)], axis=1)
    brow = jnp.zeros((8,), jnp.float32).at[4].set(b1[0] + lb1[0]).reshape(1, 8)

    ta, tb, a0, sk0 = _prep0(x_p, W0, lw0, am, bsk)
    acc0 = _edge0(ta, tb, a0, src, dst)
    t1 = _comb0(acc0[0], acc0[1], ta, tb, a0, sk0, selw, arow, wmix, brow)
    acc1 = _edge1(t1, src, dst)
    out = _final(acc1[0], acc1[1], t1)
    return out[:N]


# speedup vs baseline: 88.9193x; 2.1534x over previous
import dataclasses
import functools

import jax
import jax.numpy as jnp
from jax import lax
from jax.experimental import pallas as pl
from jax.experimental.pallas import tpu as pltpu
from jax.experimental.pallas import tpu_sc as plsc

F = 128
H = 4
C = 32
NP = 10240
TW = 80
T1W = 16
CHUNK = 128
NW = 32


def _f16(v):
    return jnp.full((16,), v, jnp.int32)


def _sc_params():
    cp = pltpu.CompilerParams()
    for field, val in (("needs_layout_passes", False),
                       ("use_tc_tiling_on_sc", False)):
        if field in pltpu.CompilerParams.__dataclass_fields__:
            cp = dataclasses.replace(cp, **{field: val})
    return cp


def _prep0_body(x_ref, w_ref, lw_ref, am_ref, bsk_ref, ta_ref, tb_ref, a_ref, s_ref):
    xb = x_ref[...]
    h = jnp.dot(xb, w_ref[...], preferred_element_type=jnp.float32)
    aa = jnp.dot(h, am_ref[...], preferred_element_type=jnp.float32)
    zp = jnp.zeros((xb.shape[0], 14), jnp.float32)
    ta_ref[...] = jnp.concatenate([h[:, 0:64], aa[:, 0:2], zp], axis=1)
    tb_ref[...] = jnp.concatenate([h[:, 64:128], aa[:, 2:4], zp], axis=1)
    a_ref[...] = jnp.concatenate([aa[:, 4:8], zp[:, 0:12]], axis=1)
    s_ref[...] = jnp.dot(xb, lw_ref[...], preferred_element_type=jnp.float32) + bsk_ref[...]


def _prep0(x_p, w0, lw0, am, bsk):
    bn = 1024
    return pl.pallas_call(
        _prep0_body,
        grid=(NP // bn,),
        in_specs=[
            pl.BlockSpec((bn, F), lambda i: (i, 0)),
            pl.BlockSpec((F, F), lambda i: (0, 0)),
            pl.BlockSpec((F, F), lambda i: (0, 0)),
            pl.BlockSpec((F, 8), lambda i: (0, 0)),
            pl.BlockSpec((1, F), lambda i: (0, 0)),
        ],
        out_specs=[
            pl.BlockSpec((bn, TW), lambda i: (i, 0)),
            pl.BlockSpec((bn, TW), lambda i: (i, 0)),
            pl.BlockSpec((bn, 16), lambda i: (i, 0)),
            pl.BlockSpec((bn, F), lambda i: (i, 0)),
        ],
        out_shape=[
            jax.ShapeDtypeStruct((NP, TW), jnp.float32),
            jax.ShapeDtypeStruct((NP, TW), jnp.float32),
            jax.ShapeDtypeStruct((NP, 16), jnp.float32),
            jax.ShapeDtypeStruct((NP, F), jnp.float32),
        ],
    )(x_p, w0, lw0, am, bsk)


def _edge0_body(ta_hbm, tb_hbm, a0_hbm, src_hbm, dst_hbm, out_hbm,
                srcbig, dstbig, rowv, av, acc, gsem0, gsem1, asem0, asem1):
    c = lax.axis_index("c")
    s = lax.axis_index("s")
    n_chunks = src_hbm.shape[0]
    per_s = n_chunks // 16
    rows_s = NP // 16
    gsems = (gsem0, gsem1)
    asems = (asem0, asem1)

    @pl.loop(0, CHUNK)
    def _(i):
        @pl.loop(0, TW, step=16)
        def _(j):
            rowv[0, i, pl.ds(j, 16)] = jnp.zeros((16,), jnp.float32)

    for k in range(rows_s // CHUNK):
        pltpu.sync_copy(rowv.at[0], acc.at[pl.ds(s * rows_s + k * CHUNK, CHUNK)])

    pltpu.sync_copy(src_hbm.at[pl.ds(s * per_s, per_s)], srcbig)
    pltpu.sync_copy(dst_hbm.at[pl.ds(s * per_s, per_s)], dstbig)
    plsc.subcore_barrier()

    def start(j, b):
        @pl.when(c == 0)
        def _():
            pltpu.async_copy(ta_hbm.at[srcbig.at[j]], rowv.at[b], gsems[b])

        @pl.when(c == 1)
        def _():
            pltpu.async_copy(tb_hbm.at[srcbig.at[j]], rowv.at[b], gsems[b])

        pltpu.async_copy(a0_hbm.at[dstbig.at[j]], av.at[b], asems[b])

    def wait(b):
        pltpu.make_async_copy(ta_hbm.at[srcbig.at[0]], rowv.at[b], gsems[b]).wait()
        pltpu.make_async_copy(a0_hbm.at[dstbig.at[0]], av.at[b], asems[b]).wait()

    def work(j, b):
        @pl.loop(0, CHUNK, step=16)
        def _(e0):
            iot = lax.iota(jnp.int32, 16) + e0
            for hl in range(2):
                a_s = plsc.load_gather(rowv.at[b], [iot, _f16(64 + hl)])
                a_d = plsc.load_gather(av.at[b], [iot, _f16(hl) + 2 * c])
                al = a_s + a_d
                al = jnp.where(al > 0, al, al * 0.2)
                ex = jnp.exp(al)
                plsc.store_scatter(rowv.at[b], [iot, _f16(64 + hl)], ex)

        @pl.loop(0, CHUNK)
        def _(e):
            exv = rowv[b, e, pl.ds(64, 16)]
            for hl in range(2):
                sc = exv[hl]
                for v in range(2):
                    off = hl * C + v * 16
                    rowv[b, e, pl.ds(off, 16)] = rowv[b, e, pl.ds(off, 16)] * sc

        pltpu.sync_copy(rowv.at[b], acc.at[dstbig.at[j]], add=True)

    start(0, 0)

    @pl.loop(0, per_s // 2)
    def _(jj):
        j0 = 2 * jj
        start(j0 + 1, 1)
        wait(0)
        work(j0, 0)

        @pl.when(j0 + 2 < per_s)
        def _():
            start(j0 + 2, 0)

        wait(1)
        work(j0 + 1, 1)

    if per_s % 2:
        wait(0)
        work(per_s - 1, 0)

    plsc.subcore_barrier()
    for k in range(rows_s // CHUNK):
        base = s * rows_s + k * CHUNK
        pltpu.sync_copy(acc.at[pl.ds(base, CHUNK)], out_hbm.at[c, pl.ds(base, CHUNK)])


def _edge0(ta, tb, a0, src2d, dst2d):
    per_s = src2d.shape[0] // 16
    mesh = plsc.VectorSubcoreMesh(core_axis_name="c", subcore_axis_name="s")
    kern = functools.partial(
        pl.kernel,
        mesh=mesh,
        out_type=jax.ShapeDtypeStruct((2, NP, TW), jnp.float32),
        scratch_types=[
            pltpu.VMEM((per_s, CHUNK), jnp.int32),
            pltpu.VMEM((per_s, CHUNK), jnp.int32),
            pltpu.VMEM((2, CHUNK, TW), jnp.float32),
            pltpu.VMEM((2, CHUNK, 16), jnp.float32),
            pltpu.VMEM_SHARED((NP, TW), jnp.float32),
            pltpu.SemaphoreType.DMA,
            pltpu.SemaphoreType.DMA,
            pltpu.SemaphoreType.DMA,
            pltpu.SemaphoreType.DMA,
        ],
        compiler_params=_sc_params(),
    )(_edge0_body)
    return kern(ta, tb, a0, src2d, dst2d)


def _comb0_body(aA_ref, aB_ref, ta_ref, tb_ref, a0_ref, sk_ref, sel_ref, ar_ref,
                wmix_ref, brow_ref, t1_ref):
    aA = aA_ref[...]
    aB = aB_ref[...]
    h = jnp.concatenate([ta_ref[...][:, 0:64], tb_ref[...][:, 0:64]], axis=1)
    asrc0 = jnp.concatenate([ta_ref[...][:, 64:66], tb_ref[...][:, 64:66]], axis=1)
    als = asrc0 + a0_ref[...][:, 0:4]
    als = jnp.where(als > 0, als, 0.2 * als)
    exs = jnp.exp(als)
    selw = sel_ref[...]
    num = jnp.concatenate([aA[:, 0:64], aB[:, 0:64]], axis=1) \
        + jnp.dot(exs, selw, preferred_element_type=jnp.float32) * h
    den4 = jnp.concatenate([aA[:, 64:66], aB[:, 64:66]], axis=1) + exs
    den = jnp.dot(den4, selw, preferred_element_type=jnp.float32) + 1e-16
    z = num / den + sk_ref[...]
    h1 = jnp.where(z > 0, z, jnp.exp(jnp.minimum(z, 0.0)) - 1.0)
    hwsk = jnp.dot(h1, wmix_ref[...], preferred_element_type=jnp.float32) + brow_ref[...]
    hw1 = hwsk[:, 0:4]
    ar = ar_ref[...]
    zp = jnp.zeros((h.shape[0], 3), jnp.float32)
    t1_ref[...] = jnp.concatenate(
        [hw1, hw1 * ar[:, 0:4], hw1 * ar[:, 4:8], hwsk[:, 4:5], zp], axis=1)


def _comb0(accA, accB, ta, tb, a0, sk0, selw, arow, wmix, brow):
    bn = 1024
    return pl.pallas_call(
        _comb0_body,
        grid=(NP // bn,),
        in_specs=[
            pl.BlockSpec((bn, TW), lambda i: (i, 0)),
            pl.BlockSpec((bn, TW), lambda i: (i, 0)),
            pl.BlockSpec((bn, TW), lambda i: (i, 0)),
            pl.BlockSpec((bn, TW), lambda i: (i, 0)),
            pl.BlockSpec((bn, 16), lambda i: (i, 0)),
            pl.BlockSpec((bn, F), lambda i: (i, 0)),
            pl.BlockSpec((4, F), lambda i: (0, 0)),
            pl.BlockSpec((1, 8), lambda i: (0, 0)),
            pl.BlockSpec((F, 8), lambda i: (0, 0)),
            pl.BlockSpec((1, 8), lambda i: (0, 0)),
        ],
        out_specs=[pl.BlockSpec((bn, T1W), lambda i: (i, 0))],
        out_shape=[jax.ShapeDtypeStruct((NP, T1W), jnp.float32)],
    )(accA, accB, ta, tb, a0, sk0, selw, arow, wmix, brow)[0]


def _edge1_body(t1_hbm, src_hbm, dst_hbm, out_hbm, srcbig, dstbig, rowv, drow,
                acc, gsem0, gsem1, dsem0, dsem1):
    c = lax.axis_index("c")
    s = lax.axis_index("s")
    wid = s * 2 + c
    per_w = src_hbm.shape[0] // NW
    rows_s = NP // 16
    gsems = (gsem0, gsem1)
    dsems = (dsem0, dsem1)

    @pl.loop(0, CHUNK)
    def _(i):
        rowv[0, i, pl.ds(0, 16)] = jnp.zeros((16,), jnp.float32)

    for k in range(rows_s // CHUNK):
        pltpu.sync_copy(rowv.at[0], acc.at[pl.ds(s * rows_s + k * CHUNK, CHUNK)])

    pltpu.sync_copy(src_hbm.at[pl.ds(wid * per_w, per_w)], srcbig)
    pltpu.sync_copy(dst_hbm.at[pl.ds(wid * per_w, per_w)], dstbig)
    plsc.subcore_barrier()

    def start(j, b):
        pltpu.async_copy(t1_hbm.at[srcbig.at[j]], rowv.at[b], gsems[b])
        pltpu.async_copy(t1_hbm.at[dstbig.at[j]], drow.at[b], dsems[b])

    def wait(b):
        pltpu.make_async_copy(t1_hbm.at[srcbig.at[0]], rowv.at[b], gsems[b]).wait()
        pltpu.make_async_copy(t1_hbm.at[dstbig.at[0]], drow.at[b], dsems[b]).wait()

    def work(j, b):
        @pl.loop(0, CHUNK, step=16)
        def _(e0):
            iot = lax.iota(jnp.int32, 16) + e0
            for hd in range(H):
                a_s = plsc.load_gather(rowv.at[b], [iot, _f16(4 + hd)])
                a_d = plsc.load_gather(drow.at[b], [iot, _f16(8 + hd)])
                al = a_s + a_d
                al = jnp.where(al > 0, al, al * 0.2)
                ex = jnp.exp(al)
                hsrc = plsc.load_gather(rowv.at[b], [iot, _f16(hd)])
                plsc.store_scatter(rowv.at[b], [iot, _f16(hd)], hsrc * ex)
                plsc.store_scatter(rowv.at[b], [iot, _f16(4 + hd)], ex)

        pltpu.sync_copy(rowv.at[b], acc.at[dstbig.at[j]], add=True)

    start(0, 0)

    @pl.loop(0, per_w // 2)
    def _(jj):
        j0 = 2 * jj
        start(j0 + 1, 1)
        wait(0)
        work(j0, 0)

        @pl.when(j0 + 2 < per_w)
        def _():
            start(j0 + 2, 0)

        wait(1)
        work(j0 + 1, 1)

    if per_w % 2:
        wait(0)
        work(per_w - 1, 0)

    plsc.subcore_barrier()
    for k in range(rows_s // CHUNK):
        base = s * rows_s + k * CHUNK
        pltpu.sync_copy(acc.at[pl.ds(base, CHUNK)], out_hbm.at[c, pl.ds(base, CHUNK)])


def _edge1(t1, src2d, dst2d):
    per_w = src2d.shape[0] // NW
    mesh = plsc.VectorSubcoreMesh(core_axis_name="c", subcore_axis_name="s")
    kern = functools.partial(
        pl.kernel,
        mesh=mesh,
        out_type=jax.ShapeDtypeStruct((2, NP, T1W), jnp.float32),
        scratch_types=[
            pltpu.VMEM((per_w, CHUNK), jnp.int32),
            pltpu.VMEM((per_w, CHUNK), jnp.int32),
            pltpu.VMEM((2, CHUNK, T1W), jnp.float32),
            pltpu.VMEM((2, CHUNK, T1W), jnp.float32),
            pltpu.VMEM_SHARED((NP, T1W), jnp.float32),
            pltpu.SemaphoreType.DMA,
            pltpu.SemaphoreType.DMA,
            pltpu.SemaphoreType.DMA,
            pltpu.SemaphoreType.DMA,
        ],
        compiler_params=_sc_params(),
    )(_edge1_body)
    return kern(t1, src2d, dst2d)


def _final_body(aA_ref, aB_ref, t1_ref, o_ref):
    t1 = t1_ref[...]
    hw1 = t1[:, 0:4]
    als = t1[:, 4:8] + t1[:, 8:12]
    als = jnp.where(als > 0, als, 0.2 * als)
    exs = jnp.exp(als)
    aA = aA_ref[...]
    aB = aB_ref[...]
    num = aA[:, 0:4] + aB[:, 0:4] + exs * hw1
    den = aA[:, 4:8] + aB[:, 4:8] + exs + 1e-16
    o_ref[...] = jnp.mean(num / den, axis=1, keepdims=True) + t1[:, 12:13]


def _final(acc1A, acc1B, t1):
    bn = 512
    return pl.pallas_call(
        _final_body,
        grid=(NP // bn,),
        in_specs=[
            pl.BlockSpec((bn, T1W), lambda i: (i, 0)),
            pl.BlockSpec((bn, T1W), lambda i: (i, 0)),
            pl.BlockSpec((bn, T1W), lambda i: (i, 0)),
        ],
        out_specs=[pl.BlockSpec((bn, 1), lambda i: (i, 0))],
        out_shape=[jax.ShapeDtypeStruct((NP, 1), jnp.float32)],
    )(acc1A, acc1B, t1)[0]


def kernel(x, edge_index, batch, W0, a_src0, a_dst0, b0, lw0, lb0,
           W1, a_src1, a_dst1, b1, lw1, lb1):
    N = x.shape[0]
    E = edge_index.shape[1]
    ep = ((E + NW * CHUNK - 1) // (NW * CHUNK)) * (NW * CHUNK)

    x_p = jnp.pad(x, ((0, NP - N), (0, 0)))
    src = jnp.pad(edge_index[0], (0, ep - E)).reshape(ep // CHUNK, CHUNK)
    dst = jnp.pad(edge_index[1], (0, ep - E), constant_values=N).reshape(ep // CHUNK, CHUNK)

    eyeH = jnp.eye(H, dtype=jnp.float32)
    am = jnp.concatenate([
        jnp.repeat(eyeH, C, axis=0) * a_src0.reshape(-1, 1),
        jnp.repeat(eyeH, C, axis=0) * a_dst0.reshape(-1, 1),
    ], axis=1)
    selw = jnp.repeat(eyeH, C, axis=1)
    bsk = (b0 + lb0).reshape(1, F)
    arow = jnp.concatenate([a_src1[:, 0], a_dst1[:, 0]]).reshape(1, 8)
    wmix = jnp.concatenate([W1, lw1, jnp.zeros((F, 3), jnp.float32)], axis=1)
    brow = jnp.zeros((8,), jnp.float32).at[4].set(b1[0] + lb1[0]).reshape(1, 8)

    ta, tb, a0, sk0 = _prep0(x_p, W0, lw0, am, bsk)
    acc0 = _edge0(ta, tb, a0, src, dst)
    t1 = _comb0(acc0[0], acc0[1], ta, tb, a0, sk0, selw, arow, wmix, brow)
    acc1 = _edge1(t1, src, dst)
    out = _final(acc1[0], acc1[1], t1)
    return out[:N]
